# baseline (speedup 1.0000x reference)
import functools

import jax
import jax.numpy as jnp
from jax import lax
from jax.experimental import pallas as pl
from jax.experimental.pallas import tpu as pltpu
from jax.experimental.pallas import tpu_sc as plsc

N = 10000
E = 320000
H = 128
SCALES = 3
LAYERS = 3
NS = [N, N // 2, N // 4, N // 8]
NPAD = [10240, 5120, 2560, 1280]

f32 = jnp.float32


def _mm_body(has_b, has_dinv, has_res, *refs):
    it = iter(refs)
    x = next(it)[...]
    W = next(it)[...]
    b = next(it)[...] if has_b else None
    dv = next(it)[...] if has_dinv else None
    res = next(it)[...] if has_res else None
    out = next(it)
    acc = jnp.dot(x, W, preferred_element_type=f32)
    if b is not None:
        acc = acc + b
    if dv is not None:
        acc = acc * dv
    if res is not None:
        acc = acc + res
    out[...] = acc


def mm(x, W, b=None, dinv=None, res=None, block=1000):
    n = x.shape[0]
    o = W.shape[1]
    grid = (pl.cdiv(n, block),)
    in_specs = [
        pl.BlockSpec((block, x.shape[1]), lambda i: (i, 0)),
        pl.BlockSpec(W.shape, lambda i: (0, 0)),
    ]
    args = [x, W]
    if b is not None:
        in_specs.append(pl.BlockSpec((1, o), lambda i: (0, 0)))
        args.append(b.reshape(1, o))
    if dinv is not None:
        in_specs.append(pl.BlockSpec((block, 1), lambda i: (i, 0)))
        args.append(dinv)
    if res is not None:
        in_specs.append(pl.BlockSpec((block, o), lambda i: (i, 0)))
        args.append(res)
    return pl.pallas_call(
        functools.partial(_mm_body, b is not None, dinv is not None, res is not None),
        grid=grid,
        in_specs=in_specs,
        out_specs=pl.BlockSpec((block, o), lambda i: (i, 0)),
        out_shape=jax.ShapeDtypeStruct((n, o), f32),
    )(*args)


def _coarse_body(kb, A_ref, g_ref, gs_ref, dv_ref, xk_ref, out_ref):
    k = pl.program_id(1)

    @pl.when(k == 0)
    def _():
        out_ref[...] = jnp.zeros_like(out_ref)

    out_ref[...] += jnp.dot(A_ref[...], g_ref[...], preferred_element_type=f32)

    @pl.when(k == kb - 1)
    def _():
        agg = dv_ref[...] * (out_ref[...] + gs_ref[...])
        out_ref[...] = jnp.maximum(agg, 0.0) + xk_ref[...]


def coarse_gcn(A, g_pad, dinv_pad, xk_pad, rblk=640, kblk=640):
    npad = A.shape[0]
    kb = npad // kblk
    grid = (npad // rblk, kb)
    return pl.pallas_call(
        functools.partial(_coarse_body, kb),
        grid=grid,
        in_specs=[
            pl.BlockSpec((rblk, kblk), lambda r, k: (r, k)),
            pl.BlockSpec((kblk, H), lambda r, k: (k, 0)),
            pl.BlockSpec((rblk, H), lambda r, k: (r, 0)),
            pl.BlockSpec((rblk, 1), lambda r, k: (r, 0)),
            pl.BlockSpec((rblk, H), lambda r, k: (r, 0)),
        ],
        out_specs=pl.BlockSpec((rblk, H), lambda r, k: (r, 0)),
        out_shape=jax.ShapeDtypeStruct((npad, H), f32),
    )(A, g_pad, g_pad, dinv_pad, xk_pad)


def _finepost_body(P, S_ref, g_ref, dv_ref, xk_ref, out_ref):
    acc = g_ref[...]
    for p in range(P):
        acc = acc + S_ref[p]
    out_ref[...] = jnp.maximum(dv_ref[...] * acc, 0.0) + xk_ref[...]


def fine_post(S, g, dinv, xk, block=2000):
    P = S.shape[0]
    n = g.shape[0]
    grid = (pl.cdiv(n, block),)
    return pl.pallas_call(
        functools.partial(_finepost_body, P),
        grid=grid,
        in_specs=[
            pl.BlockSpec((P, block, H), lambda i: (0, i, 0)),
            pl.BlockSpec((block, H), lambda i: (i, 0)),
            pl.BlockSpec((block, 1), lambda i: (i, 0)),
            pl.BlockSpec((block, H), lambda i: (i, 0)),
        ],
        out_specs=pl.BlockSpec((block, H), lambda i: (i, 0)),
        out_shape=jax.ShapeDtypeStruct((n, H), f32),
    )(S, g, dinv, xk)


def _pmean_body(x_ref, out_ref):
    x = x_ref[...]
    out_ref[...] = 0.5 * (x[:, :H] + x[:, H:])


def pmean(x, block=1000):
    m = x.shape[0] // 2
    xv = x.reshape(m, 2 * H)
    grid = (pl.cdiv(m, block),)
    return pl.pallas_call(
        _pmean_body,
        grid=grid,
        in_specs=[pl.BlockSpec((block, 2 * H), lambda i: (i, 0))],
        out_specs=pl.BlockSpec((block, H), lambda i: (i, 0)),
        out_shape=jax.ShapeDtypeStruct((m, H), f32),
    )(xv)


def _inter_body(has_mm, av_ref, *refs):
    if has_mm:
        pv_ref, W_ref, u_ref, out_ref = refs
        pv = pv_ref[...]
        W = W_ref[...]
        u = u_ref[...]
        left = jnp.dot(pv[:, :H], W, preferred_element_type=f32) + u
        right = jnp.dot(pv[:, H:], W, preferred_element_type=f32) + u
        out_ref[...] = av_ref[...] + jnp.concatenate([left, right], axis=1)
    else:
        u_ref, out_ref = refs
        u = u_ref[...]
        out_ref[...] = av_ref[...] + jnp.concatenate([u, u], axis=1)


def interleave(a, u, p=None, W=None, block=512):
    m = a.shape[0] // 2
    av = a.reshape(m, 2 * H)
    grid = (pl.cdiv(m, block),)
    in_specs = [pl.BlockSpec((block, 2 * H), lambda i: (i, 0))]
    args = [av]
    if p is not None:
        in_specs += [
            pl.BlockSpec((block, 2 * H), lambda i: (i, 0)),
            pl.BlockSpec((H, H), lambda i: (0, 0)),
        ]
        args += [p.reshape(m, 2 * H), W]
    in_specs.append(pl.BlockSpec((block, H), lambda i: (i, 0)))
    args.append(u)
    out = pl.pallas_call(
        functools.partial(_inter_body, p is not None),
        grid=grid,
        in_specs=in_specs,
        out_specs=pl.BlockSpec((block, 2 * H), lambda i: (i, 0)),
        out_shape=jax.ShapeDtypeStruct((m, 2 * H), f32),
    )(*args)
    return out.reshape(2 * m, H)


def _expand_body(F, x_ref, out_ref):
    x = x_ref[...]
    out_ref[...] = jnp.concatenate([x] * F, axis=1)


def expand(x, F, block=640):
    n = x.shape[0]
    grid = (pl.cdiv(n, block),)
    out = pl.pallas_call(
        functools.partial(_expand_body, F),
        grid=grid,
        in_specs=[pl.BlockSpec((block, H), lambda i: (i, 0))],
        out_specs=pl.BlockSpec((block, F * H), lambda i: (i, 0)),
        out_shape=jax.ShapeDtypeStruct((n, F * H), f32),
    )(x)
    return out.reshape(n * F, H)


def _cls_body(x0_ref, e1_ref, e2_ref, e3_ref, W1_ref, b1_ref, W2_ref, b2_ref, out_ref):
    z = jnp.dot(x0_ref[...], W1_ref[...], preferred_element_type=f32)
    z = z + e1_ref[...] + e2_ref[...] + e3_ref[...] + b1_ref[...]
    z = jnp.maximum(z, 0.0)
    out_ref[...] = jnp.dot(z, W2_ref[...], preferred_element_type=f32) + b2_ref[...]


def classifier(x0, e1, e2, e3, W1a, b1, W2, b2, block=2000):
    n = x0.shape[0]
    dout = W2.shape[1]
    grid = (pl.cdiv(n, block),)
    return pl.pallas_call(
        _cls_body,
        grid=grid,
        in_specs=[
            pl.BlockSpec((block, H), lambda i: (i, 0)),
            pl.BlockSpec((block, H), lambda i: (i, 0)),
            pl.BlockSpec((block, H), lambda i: (i, 0)),
            pl.BlockSpec((block, H), lambda i: (i, 0)),
            pl.BlockSpec((H, H), lambda i: (0, 0)),
            pl.BlockSpec((1, H), lambda i: (0, 0)),
            pl.BlockSpec((H, dout), lambda i: (0, 0)),
            pl.BlockSpec((1, dout), lambda i: (0, 0)),
        ],
        out_specs=pl.BlockSpec((block, dout), lambda i: (i, 0)),
        out_shape=jax.ShapeDtypeStruct((n, dout), f32),
    )(x0, e1, e2, e3, W1a, b1.reshape(1, H), W2, b2.reshape(1, dout))


def _rowsum_body(kb, A_ref, out_ref):
    k = pl.program_id(1)

    @pl.when(k == 0)
    def _():
        out_ref[...] = jnp.zeros_like(out_ref)

    out_ref[...] += jnp.sum(A_ref[...], axis=1, keepdims=True)

    @pl.when(k == kb - 1)
    def _():
        out_ref[...] = jax.lax.rsqrt(out_ref[...] + 1.0)


def rowsum_dinv(A, rblk=640, kblk=640):
    npad = A.shape[0]
    kb = npad // kblk
    grid = (npad // rblk, kb)
    return pl.pallas_call(
        functools.partial(_rowsum_body, kb),
        grid=grid,
        in_specs=[pl.BlockSpec((rblk, kblk), lambda r, k: (r, k))],
        out_specs=pl.BlockSpec((rblk, 1), lambda r, k: (r, 0)),
        out_shape=jax.ShapeDtypeStruct((npad, 1), f32),
    )(A)


SC_NC = 2
SC_NS = 16
SC_NW = SC_NC * SC_NS
NF_PAD = 10240
RPT = NF_PAD // SC_NS

FCH = 80
F_EPW = 10240
F_EPAD = F_EPW * SC_NW

CH = 80
A_BLK = 25
A_NBLK = 10
AP = [5120 * 5120, 2560 * 2560, 1280 * 1280]
ZB = 40960
ZB3 = 20480

_sc_mesh = plsc.VectorSubcoreMesh(core_axis_name="c", subcore_axis_name="s")


@functools.partial(
    pl.kernel,
    mesh=_sc_mesh,
    out_type=jax.ShapeDtypeStruct((SC_NC, NF_PAD, H), f32),
    scratch_types=[
        pltpu.VMEM((FCH,), jnp.int32),
        pltpu.VMEM((FCH,), jnp.int32),
        pltpu.VMEM((FCH, H), f32),
        pltpu.VMEM_SHARED((NF_PAD, H), f32),
        pltpu.SemaphoreType.DMA,
    ],
)
def _fine_agg_sc(g_hbm, src_hbm, dst_hbm, zrow_hbm, out_hbm,
                 sidx, didx, rows, acc, sem):
    c = lax.axis_index("c")
    s = lax.axis_index("s")
    wid = s * SC_NC + c
    pltpu.sync_copy(zrow_hbm, acc.at[pl.ds(s * RPT, RPT)])
    plsc.subcore_barrier()
    base = wid * F_EPW

    def body(i, carry):
        off = base + i * FCH
        pltpu.sync_copy(src_hbm.at[pl.ds(off, FCH)], sidx)
        pltpu.async_copy(g_hbm.at[sidx], rows, sem).wait()
        pltpu.sync_copy(dst_hbm.at[pl.ds(off, FCH)], didx)
        pltpu.sync_copy(rows, acc.at[didx], add=True)
        return carry

    lax.fori_loop(0, F_EPW // FCH, body, 0)
    plsc.subcore_barrier()
    pltpu.sync_copy(acc.at[pl.ds(s * RPT, RPT)],
                    out_hbm.at[c, pl.ds(s * RPT, RPT)])


@functools.partial(
    pl.kernel,
    mesh=_sc_mesh,
    out_type=(
        jax.ShapeDtypeStruct((AP[0] + 8,), f32),
        jax.ShapeDtypeStruct((AP[1] + 8,), f32),
        jax.ShapeDtypeStruct((AP[2] + 8,), f32),
        jax.ShapeDtypeStruct((NF_PAD,), f32),
    ),
    scratch_types=[
        pltpu.VMEM((ZB,), f32),
        pltpu.VMEM((A_BLK, CH), jnp.int32),
        pltpu.VMEM((A_BLK, CH), jnp.int32),
        pltpu.VMEM((A_BLK, CH), jnp.int32),
        pltpu.VMEM((A_BLK, CH), jnp.int32),
        pltpu.VMEM((CH,), f32),
        pltpu.VMEM_SHARED((NF_PAD,), f32),
        pltpu.SemaphoreType.DMA,
        pltpu.SemaphoreType.DMA,
        pltpu.SemaphoreType.DMA,
        pltpu.SemaphoreType.DMA,
    ],
)
def _adj_deg_sc(src5_hbm, dst5_hbm, zflat_hbm, a1, a2, a3, deg,
                zbuf, sblk, dblk, ibufa, ibufb, obuf, dacc,
                zsem, asem, bsem, dsem):
    c = lax.axis_index("c")
    s = lax.axis_index("s")
    pltpu.sync_copy(zflat_hbm.at[pl.ds(0, ZB)], zbuf)
    for j in range(CH // 16):
        obuf[pl.ds(j * 16, 16)] = jnp.full((16,), 1.0, f32)

    def load_blk(q):
        pltpu.sync_copy(src5_hbm.at[s, q], sblk)
        pltpu.sync_copy(dst5_hbm.at[s, q], dblk)

    def compute_idx(ibuf, r, k, npad, sac):
        for j in range(CH // 16):
            sl = pl.ds(j * 16, 16)
            sk = sblk[r, sl] >> k
            dk = dblk[r, sl] >> k
            ibuf[r, sl] = jnp.where(sk != dk, dk * npad + sk, sac)

    def drain(n, ref, sem):
        for _ in range(n):
            pltpu.make_async_copy(obuf, ref.at[pl.ds(0, CH)], sem).wait()

    @pl.when(c == 0)
    def _():
        def z(i, carry):
            pltpu.async_copy(zbuf, a1.at[pl.ds((s * 40 + i) * ZB, ZB)], zsem)
            return carry
        lax.fori_loop(0, 40, z, 0)

        def zd(i, carry):
            pltpu.make_async_copy(zbuf, a1.at[pl.ds(0, ZB)], zsem).wait()
            return carry
        lax.fori_loop(0, 40, zd, 0)
        plsc.subcore_barrier()

        def fire(q):
            load_blk(q)
            for r in range(A_BLK):
                compute_idx(ibufa, r, 1, 5120, AP[0])
            for r in range(A_BLK):
                pltpu.async_copy(obuf, a1.at[ibufa.at[r]], asem)

        fire(0)

        def blk(q, carry):
            drain(A_BLK, a1, asem)
            fire(q)
            return carry
        lax.fori_loop(1, A_NBLK, blk, 0)
        drain(A_BLK, a1, asem)

    @pl.when(c == 1)
    def _():
        def z2(i, carry):
            pltpu.async_copy(zbuf, a2.at[pl.ds((s * 10 + i) * ZB, ZB)], zsem)
            return carry
        lax.fori_loop(0, 10, z2, 0)

        def z3(i, carry):
            pltpu.async_copy(zbuf.at[pl.ds(0, ZB3)],
                             a3.at[pl.ds((s * 5 + i) * ZB3, ZB3)], zsem)
            return carry
        lax.fori_loop(0, 5, z3, 0)
        pltpu.sync_copy(zbuf.at[pl.ds(0, RPT)], dacc.at[pl.ds(s * RPT, RPT)])

        def zd2(i, carry):
            pltpu.make_async_copy(zbuf, a2.at[pl.ds(0, ZB)], zsem).wait()
            return carry
        lax.fori_loop(0, 10, zd2, 0)

        def zd3(i, carry):
            pltpu.make_async_copy(zbuf.at[pl.ds(0, ZB3)],
                                  a3.at[pl.ds(0, ZB3)], zsem).wait()
            return carry
        lax.fori_loop(0, 5, zd3, 0)
        plsc.subcore_barrier()

        def fire(q):
            load_blk(q)
            for r in range(A_BLK):
                compute_idx(ibufa, r, 2, 2560, AP[1])
                compute_idx(ibufb, r, 3, 1280, AP[2])
            for r in range(A_BLK):
                pltpu.async_copy(obuf, a2.at[ibufa.at[r]], asem)
                pltpu.async_copy(obuf, a3.at[ibufb.at[r]], bsem)
                pltpu.async_copy(obuf, dacc.at[dblk.at[r]], dsem, add=True)

        fire(0)

        def blk(q, carry):
            drain(A_BLK, a2, asem)
            drain(A_BLK, a3, bsem)
            drain(A_BLK, dacc, dsem)
            load_blk(q)
            for r in range(A_BLK):
                compute_idx(ibufa, r, 2, 2560, AP[1])
                compute_idx(ibufb, r, 3, 1280, AP[2])
            for r in range(A_BLK):
                pltpu.async_copy(obuf, a2.at[ibufa.at[r]], asem)
                pltpu.async_copy(obuf, a3.at[ibufb.at[r]], bsem)
                pltpu.async_copy(obuf, dacc.at[dblk.at[r]], dsem, add=True)
            return carry
        lax.fori_loop(1, A_NBLK, blk, 0)
        drain(A_BLK, a2, asem)
        drain(A_BLK, a3, bsem)
        drain(A_BLK, dacc, dsem)
        plsc.subcore_barrier()
        pltpu.sync_copy(dacc.at[pl.ds(s * RPT, RPT)],
                        deg.at[pl.ds(s * RPT, RPT)])


def fine_agg(g, src4, dst4, zrow):
    return _fine_agg_sc(g, src4, dst4, zrow)


def kernel(x, enc_W, enc_b, conv_W, conv_b, f2c_W, c2f_W, cls_W1, cls_b1,
           cls_W2, cls_b2, edge_index):
    src = edge_index[0]
    dst = edge_index[1]
    zrow = jnp.zeros((RPT, H), f32)
    npad_e = F_EPAD - E
    src4 = jnp.pad(src, (0, npad_e))
    spread = N + (jnp.arange(npad_e, dtype=jnp.int32) % (NF_PAD - N))
    dst4 = jnp.concatenate([dst, spread])
    src5 = src.reshape(SC_NS, A_NBLK, A_BLK, CH)
    dst5 = dst.reshape(SC_NS, A_NBLK, A_BLK, CH)

    a1f, a2f, a3f, cnt = _adj_deg_sc(src5, dst5, zrow.reshape(-1))
    As = [a1f[: AP[0]].reshape(5120, 5120),
          a2f[: AP[1]].reshape(2560, 2560),
          a3f[: AP[2]].reshape(1280, 1280)]
    dinvs = [jax.lax.rsqrt(cnt[:N] + 1.0).reshape(N, 1)]
    dinvs_pad = [None]
    for k in range(1, SCALES + 1):
        dp = rowsum_dinv(As[k - 1])
        dinvs_pad.append(dp)
        dinvs.append(dp[: NS[k]])

    xs = [mm(x, enc_W, b=enc_b)]
    for s in range(SCALES):
        xs.append(pmean(xs[-1]))

    for l in range(LAYERS):
        new_xs = []
        for i in range(SCALES + 1):
            g = mm(xs[i], conv_W[l, i], b=conv_b[l, i], dinv=dinvs[i])
            if i == 0:
                S = fine_agg(g, src4, dst4, zrow)
                new_xs.append(fine_post(S, g, dinvs[0], xs[0]))
            else:
                pad = ((0, NPAD[i] - NS[i]), (0, 0))
                g_pad = jnp.pad(g, pad)
                xk_pad = jnp.pad(xs[i], pad)
                out = coarse_gcn(As[i - 1], g_pad, dinvs_pad[i], xk_pad)
                new_xs.append(out[: NS[i]])
        xs = new_xs
        if l < LAYERS - 1:
            ps = [pmean(xs[i]) for i in range(SCALES)]
            us = [mm(xs[i + 1], c2f_W[l, i]) for i in range(SCALES)]
            y0 = interleave(xs[0], us[0])
            y1 = interleave(xs[1], us[1], p=ps[0], W=f2c_W[l, 0])
            y2 = interleave(xs[2], us[2], p=ps[1], W=f2c_W[l, 1])
            y3 = mm(ps[2], f2c_W[l, 2], res=xs[3])
            xs = [y0, y1, y2, y3]

    t1 = expand(mm(xs[1], cls_W1[H:2 * H]), 2)
    t2 = expand(mm(xs[2], cls_W1[2 * H:3 * H]), 4)
    t3 = expand(mm(xs[3], cls_W1[3 * H:]), 8)
    return classifier(xs[0], t1, t2, t3, cls_W1[:H], cls_b1, cls_W2, cls_b2)


# speedup vs baseline: 1.6904x; 1.6904x over previous
import functools

import jax
import jax.numpy as jnp
from jax import lax
from jax.experimental import pallas as pl
from jax.experimental.pallas import tpu as pltpu
from jax.experimental.pallas import tpu_sc as plsc

N = 10000
E = 320000
H = 128
SCALES = 3
LAYERS = 3
NS = [N, N // 2, N // 4, N // 8]
NPAD = [10240, 5120, 2560, 1280]

f32 = jnp.float32


def _mm_body(has_b, has_dinv, has_res, *refs):
    it = iter(refs)
    x = next(it)[...]
    W = next(it)[...]
    b = next(it)[...] if has_b else None
    dv = next(it)[...] if has_dinv else None
    res = next(it)[...] if has_res else None
    out = next(it)
    acc = jnp.dot(x, W, preferred_element_type=f32)
    if b is not None:
        acc = acc + b
    if dv is not None:
        acc = acc * dv
    if res is not None:
        acc = acc + res
    out[...] = acc


def mm(x, W, b=None, dinv=None, res=None, block=1000):
    n = x.shape[0]
    o = W.shape[1]
    grid = (pl.cdiv(n, block),)
    in_specs = [
        pl.BlockSpec((block, x.shape[1]), lambda i: (i, 0)),
        pl.BlockSpec(W.shape, lambda i: (0, 0)),
    ]
    args = [x, W]
    if b is not None:
        in_specs.append(pl.BlockSpec((1, o), lambda i: (0, 0)))
        args.append(b.reshape(1, o))
    if dinv is not None:
        in_specs.append(pl.BlockSpec((block, 1), lambda i: (i, 0)))
        args.append(dinv)
    if res is not None:
        in_specs.append(pl.BlockSpec((block, o), lambda i: (i, 0)))
        args.append(res)
    return pl.pallas_call(
        functools.partial(_mm_body, b is not None, dinv is not None, res is not None),
        grid=grid,
        in_specs=in_specs,
        out_specs=pl.BlockSpec((block, o), lambda i: (i, 0)),
        out_shape=jax.ShapeDtypeStruct((n, o), f32),
    )(*args)


def _coarse_body(kb, A_ref, g_ref, gs_ref, dv_ref, xk_ref, out_ref):
    k = pl.program_id(1)

    @pl.when(k == 0)
    def _():
        out_ref[...] = jnp.zeros_like(out_ref)

    out_ref[...] += jnp.dot(A_ref[...], g_ref[...], preferred_element_type=f32)

    @pl.when(k == kb - 1)
    def _():
        agg = dv_ref[...] * (out_ref[...] + gs_ref[...])
        out_ref[...] = jnp.maximum(agg, 0.0) + xk_ref[...]


def coarse_gcn(A, g_pad, dinv_pad, xk_pad, rblk=640, kblk=640):
    npad = A.shape[0]
    kb = npad // kblk
    grid = (npad // rblk, kb)
    return pl.pallas_call(
        functools.partial(_coarse_body, kb),
        grid=grid,
        in_specs=[
            pl.BlockSpec((rblk, kblk), lambda r, k: (r, k)),
            pl.BlockSpec((kblk, H), lambda r, k: (k, 0)),
            pl.BlockSpec((rblk, H), lambda r, k: (r, 0)),
            pl.BlockSpec((rblk, 1), lambda r, k: (r, 0)),
            pl.BlockSpec((rblk, H), lambda r, k: (r, 0)),
        ],
        out_specs=pl.BlockSpec((rblk, H), lambda r, k: (r, 0)),
        out_shape=jax.ShapeDtypeStruct((npad, H), f32),
    )(A, g_pad, g_pad, dinv_pad, xk_pad)


def _finepost_body(P, S_ref, g_ref, dv_ref, xk_ref, out_ref):
    acc = g_ref[...]
    for p in range(P):
        acc = acc + S_ref[p]
    out_ref[...] = jnp.maximum(dv_ref[...] * acc, 0.0) + xk_ref[...]


def fine_post(S, g, dinv, xk, block=2000):
    P = S.shape[0]
    n = g.shape[0]
    grid = (pl.cdiv(n, block),)
    return pl.pallas_call(
        functools.partial(_finepost_body, P),
        grid=grid,
        in_specs=[
            pl.BlockSpec((P, block, H), lambda i: (0, i, 0)),
            pl.BlockSpec((block, H), lambda i: (i, 0)),
            pl.BlockSpec((block, 1), lambda i: (i, 0)),
            pl.BlockSpec((block, H), lambda i: (i, 0)),
        ],
        out_specs=pl.BlockSpec((block, H), lambda i: (i, 0)),
        out_shape=jax.ShapeDtypeStruct((n, H), f32),
    )(S, g, dinv, xk)


def _pmean_body(x_ref, out_ref):
    x = x_ref[...]
    out_ref[...] = 0.5 * (x[:, :H] + x[:, H:])


def pmean(x, block=1000):
    m = x.shape[0] // 2
    xv = x.reshape(m, 2 * H)
    grid = (pl.cdiv(m, block),)
    return pl.pallas_call(
        _pmean_body,
        grid=grid,
        in_specs=[pl.BlockSpec((block, 2 * H), lambda i: (i, 0))],
        out_specs=pl.BlockSpec((block, H), lambda i: (i, 0)),
        out_shape=jax.ShapeDtypeStruct((m, H), f32),
    )(xv)


def _inter_body(has_mm, av_ref, *refs):
    if has_mm:
        pv_ref, W_ref, u_ref, out_ref = refs
        pv = pv_ref[...]
        W = W_ref[...]
        u = u_ref[...]
        left = jnp.dot(pv[:, :H], W, preferred_element_type=f32) + u
        right = jnp.dot(pv[:, H:], W, preferred_element_type=f32) + u
        out_ref[...] = av_ref[...] + jnp.concatenate([left, right], axis=1)
    else:
        u_ref, out_ref = refs
        u = u_ref[...]
        out_ref[...] = av_ref[...] + jnp.concatenate([u, u], axis=1)


def interleave(a, u, p=None, W=None, block=512):
    m = a.shape[0] // 2
    av = a.reshape(m, 2 * H)
    grid = (pl.cdiv(m, block),)
    in_specs = [pl.BlockSpec((block, 2 * H), lambda i: (i, 0))]
    args = [av]
    if p is not None:
        in_specs += [
            pl.BlockSpec((block, 2 * H), lambda i: (i, 0)),
            pl.BlockSpec((H, H), lambda i: (0, 0)),
        ]
        args += [p.reshape(m, 2 * H), W]
    in_specs.append(pl.BlockSpec((block, H), lambda i: (i, 0)))
    args.append(u)
    out = pl.pallas_call(
        functools.partial(_inter_body, p is not None),
        grid=grid,
        in_specs=in_specs,
        out_specs=pl.BlockSpec((block, 2 * H), lambda i: (i, 0)),
        out_shape=jax.ShapeDtypeStruct((m, 2 * H), f32),
    )(*args)
    return out.reshape(2 * m, H)


def _expand_body(F, x_ref, out_ref):
    x = x_ref[...]
    out_ref[...] = jnp.concatenate([x] * F, axis=1)


def expand(x, F, block=640):
    n = x.shape[0]
    grid = (pl.cdiv(n, block),)
    out = pl.pallas_call(
        functools.partial(_expand_body, F),
        grid=grid,
        in_specs=[pl.BlockSpec((block, H), lambda i: (i, 0))],
        out_specs=pl.BlockSpec((block, F * H), lambda i: (i, 0)),
        out_shape=jax.ShapeDtypeStruct((n, F * H), f32),
    )(x)
    return out.reshape(n * F, H)


def _cls_body(x0_ref, e1_ref, e2_ref, e3_ref, W1_ref, b1_ref, W2_ref, b2_ref, out_ref):
    z = jnp.dot(x0_ref[...], W1_ref[...], preferred_element_type=f32)
    z = z + e1_ref[...] + e2_ref[...] + e3_ref[...] + b1_ref[...]
    z = jnp.maximum(z, 0.0)
    out_ref[...] = jnp.dot(z, W2_ref[...], preferred_element_type=f32) + b2_ref[...]


def classifier(x0, e1, e2, e3, W1a, b1, W2, b2, block=2000):
    n = x0.shape[0]
    dout = W2.shape[1]
    grid = (pl.cdiv(n, block),)
    return pl.pallas_call(
        _cls_body,
        grid=grid,
        in_specs=[
            pl.BlockSpec((block, H), lambda i: (i, 0)),
            pl.BlockSpec((block, H), lambda i: (i, 0)),
            pl.BlockSpec((block, H), lambda i: (i, 0)),
            pl.BlockSpec((block, H), lambda i: (i, 0)),
            pl.BlockSpec((H, H), lambda i: (0, 0)),
            pl.BlockSpec((1, H), lambda i: (0, 0)),
            pl.BlockSpec((H, dout), lambda i: (0, 0)),
            pl.BlockSpec((1, dout), lambda i: (0, 0)),
        ],
        out_specs=pl.BlockSpec((block, dout), lambda i: (i, 0)),
        out_shape=jax.ShapeDtypeStruct((n, dout), f32),
    )(x0, e1, e2, e3, W1a, b1.reshape(1, H), W2, b2.reshape(1, dout))


def _rowsum_body(kb, A_ref, out_ref):
    k = pl.program_id(1)

    @pl.when(k == 0)
    def _():
        out_ref[...] = jnp.zeros_like(out_ref)

    out_ref[...] += jnp.sum(A_ref[...], axis=1, keepdims=True)

    @pl.when(k == kb - 1)
    def _():
        out_ref[...] = jax.lax.rsqrt(out_ref[...] + 1.0)


def rowsum_dinv(A, rblk=640, kblk=640):
    npad = A.shape[0]
    kb = npad // kblk
    grid = (npad // rblk, kb)
    return pl.pallas_call(
        functools.partial(_rowsum_body, kb),
        grid=grid,
        in_specs=[pl.BlockSpec((rblk, kblk), lambda r, k: (r, k))],
        out_specs=pl.BlockSpec((rblk, 1), lambda r, k: (r, 0)),
        out_shape=jax.ShapeDtypeStruct((npad, 1), f32),
    )(A)


SC_NC = 2
SC_NS = 16
SC_NW = SC_NC * SC_NS
NF_PAD = 10240
RPT = NF_PAD // SC_NS

FCH = 40
F_EPW = E // SC_NW
F_BLK = 10
F_NBLK = 25
F_NB = 4

CH = 80
A_BLK = 25
A_NBLK = 10
AP = [5120 * 5120, 2560 * 2560, 1280 * 1280]
ZB = 40960
ZB3 = 20480

_sc_mesh = plsc.VectorSubcoreMesh(core_axis_name="c", subcore_axis_name="s")


@functools.partial(
    pl.kernel,
    mesh=_sc_mesh,
    out_type=jax.ShapeDtypeStruct((SC_NC, NF_PAD, H), f32),
    scratch_types=[
        pltpu.VMEM((F_BLK, FCH), jnp.int32),
        pltpu.VMEM((F_BLK, FCH), jnp.int32),
        pltpu.VMEM((F_NB, FCH, H), f32),
        pltpu.VMEM_SHARED((NF_PAD, H), f32),
        pltpu.SemaphoreType.DMA((F_NB,)),
        pltpu.SemaphoreType.DMA((F_NB,)),
    ],
)
def _fine_agg_sc(g_hbm, src4_hbm, dst4_hbm, zrow_hbm, out_hbm,
                 sblk, dblk, rows, acc, gsem, ssem):
    c = lax.axis_index("c")
    s = lax.axis_index("s")
    wid = s * SC_NC + c
    pltpu.sync_copy(zrow_hbm, acc.at[pl.ds(s * RPT, RPT)])
    plsc.subcore_barrier()

    def load_blk(q):
        pltpu.sync_copy(src4_hbm.at[wid, q], sblk)
        pltpu.sync_copy(dst4_hbm.at[wid, q], dblk)

    def start_g(j, b):
        pltpu.async_copy(g_hbm.at[sblk.at[j]], rows.at[b], gsem.at[b])

    def wait_g(b):
        pltpu.make_async_copy(g_hbm.at[pl.ds(0, FCH)], rows.at[b],
                              gsem.at[b]).wait()

    def start_s(j, b):
        pltpu.async_copy(rows.at[b], acc.at[dblk.at[j]], ssem.at[b], add=True)

    def wait_s(b):
        pltpu.make_async_copy(rows.at[b], acc.at[pl.ds(0, FCH)],
                              ssem.at[b]).wait()

    def run_block():
        for j in range(F_BLK):
            b = j % F_NB
            wait_g(b)
            start_s(j, b)
            wait_s(b)
            if j + F_NB < F_BLK:
                start_g(j + F_NB, b)

    load_blk(0)
    for b in range(F_NB):
        start_g(b, b)

    def outer(q, carry):
        run_block()
        load_blk(q + 1)
        for b in range(F_NB):
            start_g(b, b)
        return carry

    lax.fori_loop(0, F_NBLK - 1, outer, 0)
    run_block()
    plsc.subcore_barrier()
    pltpu.sync_copy(acc.at[pl.ds(s * RPT, RPT)],
                    out_hbm.at[c, pl.ds(s * RPT, RPT)])


@functools.partial(
    pl.kernel,
    mesh=_sc_mesh,
    out_type=(
        jax.ShapeDtypeStruct((AP[0] + 8,), f32),
        jax.ShapeDtypeStruct((AP[1] + 8,), f32),
        jax.ShapeDtypeStruct((AP[2] + 8,), f32),
        jax.ShapeDtypeStruct((NF_PAD,), f32),
    ),
    scratch_types=[
        pltpu.VMEM((ZB,), f32),
        pltpu.VMEM((A_BLK, CH), jnp.int32),
        pltpu.VMEM((A_BLK, CH), jnp.int32),
        pltpu.VMEM((A_BLK, CH), jnp.int32),
        pltpu.VMEM((A_BLK, CH), jnp.int32),
        pltpu.VMEM((CH,), f32),
        pltpu.VMEM_SHARED((NF_PAD,), f32),
        pltpu.SemaphoreType.DMA,
        pltpu.SemaphoreType.DMA,
        pltpu.SemaphoreType.DMA,
        pltpu.SemaphoreType.DMA,
    ],
)
def _adj_deg_sc(src5_hbm, dst5_hbm, zflat_hbm, a1, a2, a3, deg,
                zbuf, sblk, dblk, ibufa, ibufb, obuf, dacc,
                zsem, asem, bsem, dsem):
    c = lax.axis_index("c")
    s = lax.axis_index("s")
    pltpu.sync_copy(zflat_hbm.at[pl.ds(0, ZB)], zbuf)
    for j in range(CH // 16):
        obuf[pl.ds(j * 16, 16)] = jnp.full((16,), 1.0, f32)

    def load_blk(q):
        pltpu.sync_copy(src5_hbm.at[s, q], sblk)
        pltpu.sync_copy(dst5_hbm.at[s, q], dblk)

    def compute_idx(ibuf, r, k, npad, sac):
        for j in range(CH // 16):
            sl = pl.ds(j * 16, 16)
            sk = sblk[r, sl] >> k
            dk = dblk[r, sl] >> k
            ibuf[r, sl] = jnp.where(sk != dk, dk * npad + sk, sac)

    def drain(n, ref, sem):
        for _ in range(n):
            pltpu.make_async_copy(obuf, ref.at[pl.ds(0, CH)], sem).wait()

    @pl.when(c == 0)
    def _():
        def z(i, carry):
            pltpu.async_copy(zbuf, a1.at[pl.ds((s * 40 + i) * ZB, ZB)], zsem)
            return carry
        lax.fori_loop(0, 40, z, 0)

        def zd(i, carry):
            pltpu.make_async_copy(zbuf, a1.at[pl.ds(0, ZB)], zsem).wait()
            return carry
        lax.fori_loop(0, 40, zd, 0)
        plsc.subcore_barrier()

        def fire(q):
            load_blk(q)
            for r in range(A_BLK):
                compute_idx(ibufa, r, 1, 5120, AP[0])
            for r in range(A_BLK):
                pltpu.async_copy(obuf, a1.at[ibufa.at[r]], asem)

        fire(0)

        def blk(q, carry):
            drain(A_BLK, a1, asem)
            fire(q)
            return carry
        lax.fori_loop(1, A_NBLK, blk, 0)
        drain(A_BLK, a1, asem)

    @pl.when(c == 1)
    def _():
        def z2(i, carry):
            pltpu.async_copy(zbuf, a2.at[pl.ds((s * 10 + i) * ZB, ZB)], zsem)
            return carry
        lax.fori_loop(0, 10, z2, 0)

        def z3(i, carry):
            pltpu.async_copy(zbuf.at[pl.ds(0, ZB3)],
                             a3.at[pl.ds((s * 5 + i) * ZB3, ZB3)], zsem)
            return carry
        lax.fori_loop(0, 5, z3, 0)
        pltpu.sync_copy(zbuf.at[pl.ds(0, RPT)], dacc.at[pl.ds(s * RPT, RPT)])

        def zd2(i, carry):
            pltpu.make_async_copy(zbuf, a2.at[pl.ds(0, ZB)], zsem).wait()
            return carry
        lax.fori_loop(0, 10, zd2, 0)

        def zd3(i, carry):
            pltpu.make_async_copy(zbuf.at[pl.ds(0, ZB3)],
                                  a3.at[pl.ds(0, ZB3)], zsem).wait()
            return carry
        lax.fori_loop(0, 5, zd3, 0)
        plsc.subcore_barrier()

        def fire(q):
            load_blk(q)
            for r in range(A_BLK):
                compute_idx(ibufa, r, 2, 2560, AP[1])
                compute_idx(ibufb, r, 3, 1280, AP[2])
            for r in range(A_BLK):
                pltpu.async_copy(obuf, a2.at[ibufa.at[r]], asem)
                pltpu.async_copy(obuf, a3.at[ibufb.at[r]], bsem)
                pltpu.async_copy(obuf, dacc.at[dblk.at[r]], dsem, add=True)

        fire(0)

        def blk(q, carry):
            drain(A_BLK, a2, asem)
            drain(A_BLK, a3, bsem)
            drain(A_BLK, dacc, dsem)
            load_blk(q)
            for r in range(A_BLK):
                compute_idx(ibufa, r, 2, 2560, AP[1])
                compute_idx(ibufb, r, 3, 1280, AP[2])
            for r in range(A_BLK):
                pltpu.async_copy(obuf, a2.at[ibufa.at[r]], asem)
                pltpu.async_copy(obuf, a3.at[ibufb.at[r]], bsem)
                pltpu.async_copy(obuf, dacc.at[dblk.at[r]], dsem, add=True)
            return carry
        lax.fori_loop(1, A_NBLK, blk, 0)
        drain(A_BLK, a2, asem)
        drain(A_BLK, a3, bsem)
        drain(A_BLK, dacc, dsem)
        plsc.subcore_barrier()
        pltpu.sync_copy(dacc.at[pl.ds(s * RPT, RPT)],
                        deg.at[pl.ds(s * RPT, RPT)])


def fine_agg(g, src4, dst4, zrow):
    return _fine_agg_sc(g, src4, dst4, zrow)


def kernel(x, enc_W, enc_b, conv_W, conv_b, f2c_W, c2f_W, cls_W1, cls_b1,
           cls_W2, cls_b2, edge_index):
    src = edge_index[0]
    dst = edge_index[1]
    zrow = jnp.zeros((RPT, H), f32)
    src4 = src.reshape(SC_NW, F_NBLK, F_BLK, FCH)
    dst4 = dst.reshape(SC_NW, F_NBLK, F_BLK, FCH)
    src5 = src.reshape(SC_NS, A_NBLK, A_BLK, CH)
    dst5 = dst.reshape(SC_NS, A_NBLK, A_BLK, CH)

    a1f, a2f, a3f, cnt = _adj_deg_sc(src5, dst5, zrow.reshape(-1))
    As = [a1f[: AP[0]].reshape(5120, 5120),
          a2f[: AP[1]].reshape(2560, 2560),
          a3f[: AP[2]].reshape(1280, 1280)]
    dinvs = [jax.lax.rsqrt(cnt[:N] + 1.0).reshape(N, 1)]
    dinvs_pad = [None]
    for k in range(1, SCALES + 1):
        dp = rowsum_dinv(As[k - 1])
        dinvs_pad.append(dp)
        dinvs.append(dp[: NS[k]])

    xs = [mm(x, enc_W, b=enc_b)]
    for s in range(SCALES):
        xs.append(pmean(xs[-1]))

    for l in range(LAYERS):
        new_xs = []
        for i in range(SCALES + 1):
            g = mm(xs[i], conv_W[l, i], b=conv_b[l, i], dinv=dinvs[i])
            if i == 0:
                S = fine_agg(g, src4, dst4, zrow)
                new_xs.append(fine_post(S, g, dinvs[0], xs[0]))
            else:
                pad = ((0, NPAD[i] - NS[i]), (0, 0))
                g_pad = jnp.pad(g, pad)
                xk_pad = jnp.pad(xs[i], pad)
                out = coarse_gcn(As[i - 1], g_pad, dinvs_pad[i], xk_pad)
                new_xs.append(out[: NS[i]])
        xs = new_xs
        if l < LAYERS - 1:
            ps = [pmean(xs[i]) for i in range(SCALES)]
            us = [mm(xs[i + 1], c2f_W[l, i]) for i in range(SCALES)]
            y0 = interleave(xs[0], us[0])
            y1 = interleave(xs[1], us[1], p=ps[0], W=f2c_W[l, 0])
            y2 = interleave(xs[2], us[2], p=ps[1], W=f2c_W[l, 1])
            y3 = mm(ps[2], f2c_W[l, 2], res=xs[3])
            xs = [y0, y1, y2, y3]

    t1 = expand(mm(xs[1], cls_W1[H:2 * H]), 2)
    t2 = expand(mm(xs[2], cls_W1[2 * H:3 * H]), 4)
    t3 = expand(mm(xs[3], cls_W1[3 * H:]), 8)
    return classifier(xs[0], t1, t2, t3, cls_W1[:H], cls_b1, cls_W2, cls_b2)
